# baseline (speedup 1.0000x reference)
import jax
import jax.numpy as jnp
from jax import lax
from jax.experimental import pallas as pl
from jax.experimental.pallas import tpu as pltpu
from jax.experimental.pallas import tpu_sc as plsc

N = 10000
NEIGH = 16
BOND_STEPS = 16
ANGLE_STEPS = 4
H = 128

NP = 10240
DUMP = N
NS = 16
NC = 2
SPAN = NP // NS
CH = 128
NCH = SPAN // CH
NROW = NEIGH * NCH

NDSPLIT = 4
NPD = NP // NDSPLIT
SPANH = NPD // NS

BLK = 512

F32 = jnp.float32


def _tc1_body(bond_ref, angle_ref, wlb_ref, wrb_ref, wla_ref, wra_ref,
              z_ref, rb_ref, ra_ref):
    bond = bond_ref[...]
    accl = jnp.zeros((BLK, H), F32)
    accr = jnp.zeros((BLK, H), F32)
    inv_g2b = (BOND_STEPS / (8.0 - 0.0)) ** 2
    for st in range(BOND_STEPS):
        f = 0.0 + st * (8.0 - 0.0) / (BOND_STEPS - 1)
        e = jnp.exp(-((bond - f) ** 2) * inv_g2b)
        accl = accl + jnp.dot(e, wlb_ref[st], preferred_element_type=F32, precision=jax.lax.Precision.HIGHEST)
        accr = accr + jnp.dot(e, wrb_ref[st], preferred_element_type=F32, precision=jax.lax.Precision.HIGHEST)
    z_ref[0] = accl
    rb_ref[...] = accr

    ang = angle_ref[...]
    accl = jnp.zeros((BLK, H), F32)
    accr = jnp.zeros((BLK, H), F32)
    inv_g2a = (ANGLE_STEPS / (1.0 - (-1.0))) ** 2
    for st in range(ANGLE_STEPS):
        f = -1.0 + st * (1.0 - (-1.0)) / (ANGLE_STEPS - 1)
        e = jnp.exp(-((ang - f) ** 2) * inv_g2a)
        accl = accl + jnp.dot(e, wla_ref[st], preferred_element_type=F32, precision=jax.lax.Precision.HIGHEST)
        accr = accr + jnp.dot(e, wra_ref[st], preferred_element_type=F32, precision=jax.lax.Precision.HIGHEST)
    z_ref[1] = accl
    ra_ref[...] = accr


def _tc1(bond_p, angle_p, wlb_s, wrb_s, wla_s, wra_s):
    grid = (NP // BLK,)
    return pl.pallas_call(
        _tc1_body,
        grid=grid,
        in_specs=[
            pl.BlockSpec((BLK, NEIGH), lambda i: (i, 0)),
            pl.BlockSpec((BLK, NEIGH * NEIGH), lambda i: (i, 0)),
            pl.BlockSpec((BOND_STEPS, NEIGH, H), lambda i: (0, 0, 0)),
            pl.BlockSpec((BOND_STEPS, NEIGH, H), lambda i: (0, 0, 0)),
            pl.BlockSpec((ANGLE_STEPS, NEIGH * NEIGH, H), lambda i: (0, 0, 0)),
            pl.BlockSpec((ANGLE_STEPS, NEIGH * NEIGH, H), lambda i: (0, 0, 0)),
        ],
        out_specs=[
            pl.BlockSpec((2, BLK, H), lambda i: (0, i, 0)),
            pl.BlockSpec((BLK, H), lambda i: (i, 0)),
            pl.BlockSpec((BLK, H), lambda i: (i, 0)),
        ],
        out_shape=[
            jax.ShapeDtypeStruct((2, NP, H), F32),
            jax.ShapeDtypeStruct((NP, H), F32),
            jax.ShapeDtypeStruct((NP, H), F32),
        ],
    )(bond_p, angle_p, wlb_s, wrb_s, wla_s, wra_s)


def _sc_mesh():
    return plsc.VectorSubcoreMesh(core_axis_name="c", subcore_axis_name="s",
                                  num_cores=NC, num_subcores=NS)


def _sc_scatter_body(z_hbm, idx_hbm, zeros_hbm, s_hbm, idx_v, z_v, hist):
    c = lax.axis_index("c")
    s = lax.axis_index("s")
    for p in range(NDSPLIT):
        span_h = pl.ds(s * SPANH, SPANH)
        pltpu.sync_copy(zeros_hbm.at[span_h], hist.at[span_h])
        pltpu.sync_copy(idx_hbm.at[p, pl.ds(s * NROW, NROW)], idx_v)
        plsc.subcore_barrier()
        for j in range(NCH):
            pltpu.sync_copy(z_hbm.at[c, pl.ds(s * SPAN + j * CH, CH)], z_v)
            for k in range(NEIGH):
                pltpu.sync_copy(
                    z_v,
                    hist.at[plsc.Indices(idx_v.at[j * NEIGH + k],
                                         ignored_value=-1)],
                    add=True)
        plsc.subcore_barrier()
        pltpu.sync_copy(hist.at[span_h],
                        s_hbm.at[c, pl.ds(p * NPD + s * SPANH, SPANH)])


def _make_sc_scatter():
    return pl.kernel(
        _sc_scatter_body,
        out_type=jax.ShapeDtypeStruct((NC, NP, H), F32),
        mesh=_sc_mesh(),
        compiler_params=pltpu.CompilerParams(needs_layout_passes=False),
        scratch_types=[
            pltpu.VMEM((NROW, CH), jnp.int32),
            pltpu.VMEM((CH, H), F32),
            pltpu.VMEM_SHARED((NPD, H), F32),
        ],
    )


def _sc_cnt_body(idxr_hbm, zeros_hbm, ident_hbm, cnt_hbm,
                 idx_v, ident_v, hist2d, cnt_sh):
    s = lax.axis_index("s")
    zvec = jnp.zeros((16,), F32)
    ovec = jnp.ones((16,), F32)
    for i in range(NROW):
        for l in range(CH // 16):
            hist2d[i, pl.ds(l * 16, 16)] = zvec
    pltpu.sync_copy(idxr_hbm.at[pl.ds(s * NROW, NROW)], idx_v)
    pltpu.sync_copy(ident_hbm, ident_v)
    for i in range(NROW):
        for l in range(CH // 16):
            iv = idx_v[i, pl.ds(l * 16, 16)]
            hi = lax.shift_right_logical(iv, 7)
            lo = jnp.bitwise_and(iv, 127)
            plsc.addupdate_scatter(hist2d, [hi, lo], ovec)

    @pl.when(s == 0)
    def _():
        pltpu.sync_copy(zeros_hbm.at[pl.ds(0, NROW)], cnt_sh)
    plsc.subcore_barrier()
    for t in range(NS):
        @pl.when(s == t)
        def _():
            pltpu.sync_copy(hist2d, cnt_sh.at[ident_v], add=True)
        plsc.subcore_barrier()

    c = lax.axis_index("c")

    @pl.when(s == 0)
    def _():
        pltpu.sync_copy(cnt_sh, cnt_hbm.at[c])


def _make_sc_cnt():
    return pl.kernel(
        _sc_cnt_body,
        out_type=jax.ShapeDtypeStruct((NC, NROW, CH), F32),
        mesh=_sc_mesh(),
        compiler_params=pltpu.CompilerParams(needs_layout_passes=False),
        scratch_types=[
            pltpu.VMEM((NROW, CH), jnp.int32),
            pltpu.VMEM((NROW,), jnp.int32),
            pltpu.VMEM((NROW, CH), F32),
            pltpu.VMEM_SHARED((NROW, CH), F32),
        ],
    )


def _tc2_body(s_ref, cnt_ref, rb_ref, ra_ref,
              wlb_ref, wrb_ref, wla_ref, wra_ref,
              blb_ref, bla_ref, z_ref, r2b_ref, r2a_ref):
    inv = 1.0 / jnp.maximum(cnt_ref[...][:, 0:1], 1.0)
    b1 = jnp.maximum(s_ref[0] * inv + blb_ref[...] + rb_ref[...], 0.0)
    a1 = jnp.maximum(s_ref[1] * inv + bla_ref[...] + ra_ref[...], 0.0)
    z_ref[0] = jnp.dot(b1, wlb_ref[...], preferred_element_type=F32, precision=jax.lax.Precision.HIGHEST)
    r2b_ref[...] = jnp.dot(b1, wrb_ref[...], preferred_element_type=F32, precision=jax.lax.Precision.HIGHEST)
    z_ref[1] = jnp.dot(a1, wla_ref[...], preferred_element_type=F32, precision=jax.lax.Precision.HIGHEST)
    r2a_ref[...] = jnp.dot(a1, wra_ref[...], preferred_element_type=F32, precision=jax.lax.Precision.HIGHEST)


def _tc2(s1, cnt16, rb, ra, wlb_t, wrb_t, wla_t, wra_t, blb, bla):
    grid = (NP // BLK,)
    row_spec = pl.BlockSpec((BLK, H), lambda i: (i, 0))
    s_spec = pl.BlockSpec((2, BLK, H), lambda i: (0, i, 0))
    w_spec = pl.BlockSpec((H, H), lambda i: (0, 0))
    b_spec = pl.BlockSpec((1, H), lambda i: (0, 0))
    return pl.pallas_call(
        _tc2_body,
        grid=grid,
        in_specs=[s_spec,
                  pl.BlockSpec((BLK, NEIGH), lambda i: (i, 0)),
                  row_spec, row_spec, w_spec, w_spec, w_spec, w_spec,
                  b_spec, b_spec],
        out_specs=[s_spec, row_spec, row_spec],
        out_shape=[jax.ShapeDtypeStruct((2, NP, H), F32),
                   jax.ShapeDtypeStruct((NP, H), F32),
                   jax.ShapeDtypeStruct((NP, H), F32)],
    )(s1, cnt16, rb, ra, wlb_t, wrb_t, wla_t, wra_t, blb, bla)


def _tc3_body(s_ref, cnt_ref, rb_ref, ra_ref,
              blb_ref, bla_ref, wm1_ref, wm2_ref, bm_ref, wf_ref, bf_ref,
              out_ref):
    inv = 1.0 / jnp.maximum(cnt_ref[...][:, 0:1], 1.0)
    b2 = jnp.maximum(s_ref[0] * inv + blb_ref[...] + rb_ref[...], 0.0)
    a2 = jnp.maximum(s_ref[1] * inv + bla_ref[...] + ra_ref[...], 0.0)
    crys = (jnp.dot(b2, wm1_ref[...], preferred_element_type=F32, precision=jax.lax.Precision.HIGHEST)
            + jnp.dot(a2, wm2_ref[...], preferred_element_type=F32, precision=jax.lax.Precision.HIGHEST)
            + bm_ref[...])
    out_ref[...] = jnp.dot(crys, wf_ref[...],
                           preferred_element_type=F32, precision=jax.lax.Precision.HIGHEST) + bf_ref[...]


def _tc3(s2, cnt16, rb, ra, blb, bla, wm1, wm2, bm2d, wf_pad, bf_pad):
    grid = (NP // BLK,)
    row_spec = pl.BlockSpec((BLK, H), lambda i: (i, 0))
    return pl.pallas_call(
        _tc3_body,
        grid=grid,
        in_specs=[pl.BlockSpec((2, BLK, H), lambda i: (0, i, 0)),
                  pl.BlockSpec((BLK, NEIGH), lambda i: (i, 0)),
                  row_spec, row_spec,
                  pl.BlockSpec((1, H), lambda i: (0, 0)),
                  pl.BlockSpec((1, H), lambda i: (0, 0)),
                  pl.BlockSpec((H, 2 * H), lambda i: (0, 0)),
                  pl.BlockSpec((H, 2 * H), lambda i: (0, 0)),
                  pl.BlockSpec((1, 2 * H), lambda i: (0, 0)),
                  pl.BlockSpec((2 * H, H), lambda i: (0, 0)),
                  pl.BlockSpec((1, H), lambda i: (0, 0))],
        out_specs=[row_spec],
        out_shape=[jax.ShapeDtypeStruct((NP, H), F32)],
    )(s2, cnt16, rb, ra, blb, bla, wm1, wm2, bm2d, wf_pad, bf_pad)[0]


def kernel(bond_fea, angle_fea, species, nbr_idx, crys_idx,
           Wl1b, bl1b, Wr1b, Wl1a, bl1a, Wr1a,
           Wl2b, bl2b, Wr2b, Wl2a, bl2a, Wr2a,
           Wm, bm, Wf, bf):
    del species, crys_idx
    pad = NP - N
    bond_p = jnp.pad(bond_fea, ((0, pad), (0, 0)))
    angle_p = jnp.pad(angle_fea.reshape(N, NEIGH * NEIGH), ((0, pad), (0, 0)))

    wlb_s = jnp.transpose(Wl1b.T.reshape(NEIGH, BOND_STEPS, H), (1, 0, 2))
    wrb_s = jnp.transpose(Wr1b.T.reshape(NEIGH, BOND_STEPS, H), (1, 0, 2))
    wla_s = jnp.transpose(
        Wl1a.T.reshape(NEIGH * NEIGH, ANGLE_STEPS, H), (1, 0, 2))
    wra_s = jnp.transpose(
        Wr1a.T.reshape(NEIGH * NEIGH, ANGLE_STEPS, H), (1, 0, 2))

    nbr_ext = jnp.concatenate(
        [nbr_idx.astype(jnp.int32),
         jnp.full((pad, NEIGH), DUMP, jnp.int32)], axis=0)
    idx_raw = jnp.transpose(
        nbr_ext.T.reshape(NEIGH, NS, NCH, CH),
        (1, 2, 0, 3)).reshape(NS * NROW, CH)
    pad_mask = idx_raw >= N
    quarters = []
    for p in range(NDSPLIT):
        rel = idx_raw - p * NPD
        ok = (rel >= 0) & (rel < NPD) & ~pad_mask
        quarters.append(jnp.where(ok, rel, -1))
    idx_split = jnp.stack(quarters)

    zeros_hbm = jnp.zeros((NPD, H), F32)
    ident_hbm = jnp.arange(NROW, dtype=jnp.int32)

    cnt2 = _make_sc_cnt()(idx_raw, zeros_hbm, ident_hbm)
    cnt16 = jnp.broadcast_to(cnt2[0].reshape(NP, 1), (NP, NEIGH))

    z1, rb, ra = _tc1(bond_p, angle_p, wlb_s, wrb_s, wla_s, wra_s)

    s1 = _make_sc_scatter()(z1, idx_split, zeros_hbm)

    z2, r2b, r2a = _tc2(s1, cnt16, rb, ra,
                        Wl2b.T, Wr2b.T, Wl2a.T, Wr2a.T,
                        bl1b.reshape(1, H), bl1a.reshape(1, H))

    s2 = _make_sc_scatter()(z2, idx_split, zeros_hbm)

    wf_pad = jnp.pad(Wf.T, ((0, 0), (0, H - 2)))
    out_pad = _tc3(s2, cnt16, r2b, r2a,
                   bl2b.reshape(1, H), bl2a.reshape(1, H),
                   Wm.T[:H], Wm.T[H:], bm.reshape(1, 2 * H),
                   wf_pad, jnp.pad(bf, (0, H - 2)).reshape(1, H))
    return out_pad[:N, :2]


# speedup vs baseline: 1.4326x; 1.4326x over previous
import jax
import jax.numpy as jnp
from jax import lax
from jax.experimental import pallas as pl
from jax.experimental.pallas import tpu as pltpu
from jax.experimental.pallas import tpu_sc as plsc

N = 10000
NEIGH = 16
BOND_STEPS = 16
ANGLE_STEPS = 4
H = 128

NP = 10240
DUMP = N
NS = 16
NC = 2
SPAN = NP // NS
CH = 128
NCH = SPAN // CH
NROW = NEIGH * NCH

NDSPLIT = 2
NPD = NP // NDSPLIT
SPANH = NPD // NS

BLK = 512

F32 = jnp.float32


def _tc1_body(bond_ref, angle_ref, wlb_ref, wrb_ref, wla_ref, wra_ref,
              z_ref, rb_ref, ra_ref):
    bond = bond_ref[...]
    accl = jnp.zeros((BLK, H), F32)
    accr = jnp.zeros((BLK, H), F32)
    inv_g2b = (BOND_STEPS / (8.0 - 0.0)) ** 2
    for st in range(BOND_STEPS):
        f = 0.0 + st * (8.0 - 0.0) / (BOND_STEPS - 1)
        e = jnp.exp(-((bond - f) ** 2) * inv_g2b)
        accl = accl + jnp.dot(e, wlb_ref[st], preferred_element_type=F32, precision=jax.lax.Precision.HIGHEST)
        accr = accr + jnp.dot(e, wrb_ref[st], preferred_element_type=F32, precision=jax.lax.Precision.HIGHEST)
    z_ref[0] = accl
    rb_ref[...] = accr

    ang = angle_ref[...]
    accl = jnp.zeros((BLK, H), F32)
    accr = jnp.zeros((BLK, H), F32)
    inv_g2a = (ANGLE_STEPS / (1.0 - (-1.0))) ** 2
    for st in range(ANGLE_STEPS):
        f = -1.0 + st * (1.0 - (-1.0)) / (ANGLE_STEPS - 1)
        e = jnp.exp(-((ang - f) ** 2) * inv_g2a)
        accl = accl + jnp.dot(e, wla_ref[st], preferred_element_type=F32, precision=jax.lax.Precision.HIGHEST)
        accr = accr + jnp.dot(e, wra_ref[st], preferred_element_type=F32, precision=jax.lax.Precision.HIGHEST)
    z_ref[1] = accl
    ra_ref[...] = accr


def _tc1(bond_p, angle_p, wlb_s, wrb_s, wla_s, wra_s):
    grid = (NP // BLK,)
    return pl.pallas_call(
        _tc1_body,
        grid=grid,
        in_specs=[
            pl.BlockSpec((BLK, NEIGH), lambda i: (i, 0)),
            pl.BlockSpec((BLK, NEIGH * NEIGH), lambda i: (i, 0)),
            pl.BlockSpec((BOND_STEPS, NEIGH, H), lambda i: (0, 0, 0)),
            pl.BlockSpec((BOND_STEPS, NEIGH, H), lambda i: (0, 0, 0)),
            pl.BlockSpec((ANGLE_STEPS, NEIGH * NEIGH, H), lambda i: (0, 0, 0)),
            pl.BlockSpec((ANGLE_STEPS, NEIGH * NEIGH, H), lambda i: (0, 0, 0)),
        ],
        out_specs=[
            pl.BlockSpec((2, BLK, H), lambda i: (0, i, 0)),
            pl.BlockSpec((BLK, H), lambda i: (i, 0)),
            pl.BlockSpec((BLK, H), lambda i: (i, 0)),
        ],
        out_shape=[
            jax.ShapeDtypeStruct((2, NP, H), F32),
            jax.ShapeDtypeStruct((NP, H), F32),
            jax.ShapeDtypeStruct((NP, H), F32),
        ],
    )(bond_p, angle_p, wlb_s, wrb_s, wla_s, wra_s)


def _sc_mesh():
    return plsc.VectorSubcoreMesh(core_axis_name="c", subcore_axis_name="s",
                                  num_cores=NC, num_subcores=NS)


def _sc_scatter_body(z_hbm, idx_hbm, zeros_hbm, s_hbm, idx_v, z_v, sem, hist):
    c = lax.axis_index("c")
    s = lax.axis_index("s")
    for p in range(NDSPLIT):
        span_h = pl.ds(s * SPANH, SPANH)
        pltpu.sync_copy(zeros_hbm.at[span_h], hist.at[span_h])
        pltpu.sync_copy(idx_hbm.at[p, pl.ds(s * NROW, NROW)], idx_v)
        plsc.subcore_barrier()
        for j in range(NCH):
            pltpu.sync_copy(z_hbm.at[c, pl.ds(s * SPAN + j * CH, CH)], z_v)
            descs = [
                pltpu.async_copy(
                    z_v,
                    hist.at[plsc.Indices(idx_v.at[j * NEIGH + k],
                                         ignored_value=-1)],
                    sem, add=True)
                for k in range(NEIGH)
            ]
            for dsc in descs:
                dsc.wait()
        plsc.subcore_barrier()
        pltpu.sync_copy(hist.at[span_h],
                        s_hbm.at[c, pl.ds(p * NPD + s * SPANH, SPANH)])


def _make_sc_scatter():
    return pl.kernel(
        _sc_scatter_body,
        out_type=jax.ShapeDtypeStruct((NC, NP, H), F32),
        mesh=_sc_mesh(),
        compiler_params=pltpu.CompilerParams(needs_layout_passes=False),
        scratch_types=[
            pltpu.VMEM((NROW, CH), jnp.int32),
            pltpu.VMEM((CH, H), F32),
            pltpu.SemaphoreType.DMA,
            pltpu.VMEM_SHARED((NPD, H), F32),
        ],
    )


def _sc_cnt_body(idxr_hbm, zeros_hbm, ident_hbm, cnt_hbm,
                 idx_v, ident_v, hist2d, cnt_sh):
    s = lax.axis_index("s")
    zvec = jnp.zeros((16,), F32)
    ovec = jnp.ones((16,), F32)
    for i in range(NROW):
        for l in range(CH // 16):
            hist2d[i, pl.ds(l * 16, 16)] = zvec
    pltpu.sync_copy(idxr_hbm.at[pl.ds(s * NROW, NROW)], idx_v)
    pltpu.sync_copy(ident_hbm, ident_v)
    for i in range(NROW):
        for l in range(CH // 16):
            iv = idx_v[i, pl.ds(l * 16, 16)]
            hi = lax.shift_right_logical(iv, 7)
            lo = jnp.bitwise_and(iv, 127)
            plsc.addupdate_scatter(hist2d, [hi, lo], ovec)

    @pl.when(s == 0)
    def _():
        pltpu.sync_copy(zeros_hbm.at[pl.ds(0, NROW)], cnt_sh)
    plsc.subcore_barrier()
    for t in range(NS):
        @pl.when(s == t)
        def _():
            pltpu.sync_copy(hist2d, cnt_sh.at[ident_v], add=True)
        plsc.subcore_barrier()

    c = lax.axis_index("c")

    @pl.when(s == 0)
    def _():
        pltpu.sync_copy(cnt_sh, cnt_hbm.at[c])


def _make_sc_cnt():
    return pl.kernel(
        _sc_cnt_body,
        out_type=jax.ShapeDtypeStruct((NC, NROW, CH), F32),
        mesh=_sc_mesh(),
        compiler_params=pltpu.CompilerParams(needs_layout_passes=False),
        scratch_types=[
            pltpu.VMEM((NROW, CH), jnp.int32),
            pltpu.VMEM((NROW,), jnp.int32),
            pltpu.VMEM((NROW, CH), F32),
            pltpu.VMEM_SHARED((NROW, CH), F32),
        ],
    )


def _tc2_body(s_ref, cnt_ref, rb_ref, ra_ref,
              wlb_ref, wrb_ref, wla_ref, wra_ref,
              blb_ref, bla_ref, z_ref, r2b_ref, r2a_ref):
    inv = 1.0 / jnp.maximum(cnt_ref[...][:, 0:1], 1.0)
    b1 = jnp.maximum(s_ref[0] * inv + blb_ref[...] + rb_ref[...], 0.0)
    a1 = jnp.maximum(s_ref[1] * inv + bla_ref[...] + ra_ref[...], 0.0)
    z_ref[0] = jnp.dot(b1, wlb_ref[...], preferred_element_type=F32, precision=jax.lax.Precision.HIGHEST)
    r2b_ref[...] = jnp.dot(b1, wrb_ref[...], preferred_element_type=F32, precision=jax.lax.Precision.HIGHEST)
    z_ref[1] = jnp.dot(a1, wla_ref[...], preferred_element_type=F32, precision=jax.lax.Precision.HIGHEST)
    r2a_ref[...] = jnp.dot(a1, wra_ref[...], preferred_element_type=F32, precision=jax.lax.Precision.HIGHEST)


def _tc2(s1, cnt16, rb, ra, wlb_t, wrb_t, wla_t, wra_t, blb, bla):
    grid = (NP // BLK,)
    row_spec = pl.BlockSpec((BLK, H), lambda i: (i, 0))
    s_spec = pl.BlockSpec((2, BLK, H), lambda i: (0, i, 0))
    w_spec = pl.BlockSpec((H, H), lambda i: (0, 0))
    b_spec = pl.BlockSpec((1, H), lambda i: (0, 0))
    return pl.pallas_call(
        _tc2_body,
        grid=grid,
        in_specs=[s_spec,
                  pl.BlockSpec((BLK, NEIGH), lambda i: (i, 0)),
                  row_spec, row_spec, w_spec, w_spec, w_spec, w_spec,
                  b_spec, b_spec],
        out_specs=[s_spec, row_spec, row_spec],
        out_shape=[jax.ShapeDtypeStruct((2, NP, H), F32),
                   jax.ShapeDtypeStruct((NP, H), F32),
                   jax.ShapeDtypeStruct((NP, H), F32)],
    )(s1, cnt16, rb, ra, wlb_t, wrb_t, wla_t, wra_t, blb, bla)


def _tc3_body(s_ref, cnt_ref, rb_ref, ra_ref,
              blb_ref, bla_ref, wm1_ref, wm2_ref, bm_ref, wf_ref, bf_ref,
              out_ref):
    inv = 1.0 / jnp.maximum(cnt_ref[...][:, 0:1], 1.0)
    b2 = jnp.maximum(s_ref[0] * inv + blb_ref[...] + rb_ref[...], 0.0)
    a2 = jnp.maximum(s_ref[1] * inv + bla_ref[...] + ra_ref[...], 0.0)
    crys = (jnp.dot(b2, wm1_ref[...], preferred_element_type=F32, precision=jax.lax.Precision.HIGHEST)
            + jnp.dot(a2, wm2_ref[...], preferred_element_type=F32, precision=jax.lax.Precision.HIGHEST)
            + bm_ref[...])
    out_ref[...] = jnp.dot(crys, wf_ref[...],
                           preferred_element_type=F32, precision=jax.lax.Precision.HIGHEST) + bf_ref[...]


def _tc3(s2, cnt16, rb, ra, blb, bla, wm1, wm2, bm2d, wf_pad, bf_pad):
    grid = (NP // BLK,)
    row_spec = pl.BlockSpec((BLK, H), lambda i: (i, 0))
    return pl.pallas_call(
        _tc3_body,
        grid=grid,
        in_specs=[pl.BlockSpec((2, BLK, H), lambda i: (0, i, 0)),
                  pl.BlockSpec((BLK, NEIGH), lambda i: (i, 0)),
                  row_spec, row_spec,
                  pl.BlockSpec((1, H), lambda i: (0, 0)),
                  pl.BlockSpec((1, H), lambda i: (0, 0)),
                  pl.BlockSpec((H, 2 * H), lambda i: (0, 0)),
                  pl.BlockSpec((H, 2 * H), lambda i: (0, 0)),
                  pl.BlockSpec((1, 2 * H), lambda i: (0, 0)),
                  pl.BlockSpec((2 * H, H), lambda i: (0, 0)),
                  pl.BlockSpec((1, H), lambda i: (0, 0))],
        out_specs=[row_spec],
        out_shape=[jax.ShapeDtypeStruct((NP, H), F32)],
    )(s2, cnt16, rb, ra, blb, bla, wm1, wm2, bm2d, wf_pad, bf_pad)[0]


def kernel(bond_fea, angle_fea, species, nbr_idx, crys_idx,
           Wl1b, bl1b, Wr1b, Wl1a, bl1a, Wr1a,
           Wl2b, bl2b, Wr2b, Wl2a, bl2a, Wr2a,
           Wm, bm, Wf, bf):
    del species, crys_idx
    pad = NP - N
    bond_p = jnp.pad(bond_fea, ((0, pad), (0, 0)))
    angle_p = jnp.pad(angle_fea.reshape(N, NEIGH * NEIGH), ((0, pad), (0, 0)))

    wlb_s = jnp.transpose(Wl1b.T.reshape(NEIGH, BOND_STEPS, H), (1, 0, 2))
    wrb_s = jnp.transpose(Wr1b.T.reshape(NEIGH, BOND_STEPS, H), (1, 0, 2))
    wla_s = jnp.transpose(
        Wl1a.T.reshape(NEIGH * NEIGH, ANGLE_STEPS, H), (1, 0, 2))
    wra_s = jnp.transpose(
        Wr1a.T.reshape(NEIGH * NEIGH, ANGLE_STEPS, H), (1, 0, 2))

    nbr_ext = jnp.concatenate(
        [nbr_idx.astype(jnp.int32),
         jnp.full((pad, NEIGH), DUMP, jnp.int32)], axis=0)
    idx_raw = jnp.transpose(
        nbr_ext.T.reshape(NEIGH, NS, NCH, CH),
        (1, 2, 0, 3)).reshape(NS * NROW, CH)
    pad_mask = idx_raw >= N
    quarters = []
    for p in range(NDSPLIT):
        rel = idx_raw - p * NPD
        ok = (rel >= 0) & (rel < NPD) & ~pad_mask
        quarters.append(jnp.where(ok, rel, -1))
    idx_split = jnp.stack(quarters)

    zeros_hbm = jnp.zeros((NPD, H), F32)
    ident_hbm = jnp.arange(NROW, dtype=jnp.int32)

    cnt2 = _make_sc_cnt()(idx_raw, zeros_hbm, ident_hbm)
    cnt16 = jnp.broadcast_to(cnt2[0].reshape(NP, 1), (NP, NEIGH))

    z1, rb, ra = _tc1(bond_p, angle_p, wlb_s, wrb_s, wla_s, wra_s)

    s1 = _make_sc_scatter()(z1, idx_split, zeros_hbm)

    z2, r2b, r2a = _tc2(s1, cnt16, rb, ra,
                        Wl2b.T, Wr2b.T, Wl2a.T, Wr2a.T,
                        bl1b.reshape(1, H), bl1a.reshape(1, H))

    s2 = _make_sc_scatter()(z2, idx_split, zeros_hbm)

    wf_pad = jnp.pad(Wf.T, ((0, 0), (0, H - 2)))
    out_pad = _tc3(s2, cnt16, r2b, r2a,
                   bl2b.reshape(1, H), bl2a.reshape(1, H),
                   Wm.T[:H], Wm.T[H:], bm.reshape(1, 2 * H),
                   wf_pad, jnp.pad(bf, (0, H - 2)).reshape(1, H))
    return out_pad[:N, :2]


# speedup vs baseline: 1.4903x; 1.0403x over previous
import jax
import jax.numpy as jnp
from jax import lax
from jax.experimental import pallas as pl
from jax.experimental.pallas import tpu as pltpu
from jax.experimental.pallas import tpu_sc as plsc

N = 10000
NEIGH = 16
BOND_STEPS = 16
ANGLE_STEPS = 4
H = 128

NP = 10240
DUMP = N
NS = 16
NC = 2
SPAN = NP // NS
CH = 128
NCH = SPAN // CH
NROW = NEIGH * NCH

NDSPLIT = 2
NPD = NP // NDSPLIT
SPANH = NPD // NS

BLK = 512

F32 = jnp.float32


def _tc1_body(bond_ref, angle_ref, wlb_ref, wrb_ref, wla_ref, wra_ref,
              z_ref, rb_ref, ra_ref):
    bond = bond_ref[...]
    accl = jnp.zeros((BLK, H), F32)
    accr = jnp.zeros((BLK, H), F32)
    inv_g2b = (BOND_STEPS / (8.0 - 0.0)) ** 2
    for st in range(BOND_STEPS):
        f = 0.0 + st * (8.0 - 0.0) / (BOND_STEPS - 1)
        e = jnp.exp(-((bond - f) ** 2) * inv_g2b)
        accl = accl + jnp.dot(e, wlb_ref[st], preferred_element_type=F32, precision=jax.lax.Precision.HIGHEST)
        accr = accr + jnp.dot(e, wrb_ref[st], preferred_element_type=F32, precision=jax.lax.Precision.HIGHEST)
    z_ref[0] = accl
    rb_ref[...] = accr

    ang = angle_ref[...]
    accl = jnp.zeros((BLK, H), F32)
    accr = jnp.zeros((BLK, H), F32)
    inv_g2a = (ANGLE_STEPS / (1.0 - (-1.0))) ** 2
    for st in range(ANGLE_STEPS):
        f = -1.0 + st * (1.0 - (-1.0)) / (ANGLE_STEPS - 1)
        e = jnp.exp(-((ang - f) ** 2) * inv_g2a)
        accl = accl + jnp.dot(e, wla_ref[st], preferred_element_type=F32, precision=jax.lax.Precision.HIGHEST)
        accr = accr + jnp.dot(e, wra_ref[st], preferred_element_type=F32, precision=jax.lax.Precision.HIGHEST)
    z_ref[1] = accl
    ra_ref[...] = accr


def _tc1(bond_p, angle_p, wlb_s, wrb_s, wla_s, wra_s):
    grid = (NP // BLK,)
    return pl.pallas_call(
        _tc1_body,
        grid=grid,
        in_specs=[
            pl.BlockSpec((BLK, NEIGH), lambda i: (i, 0)),
            pl.BlockSpec((BLK, NEIGH * NEIGH), lambda i: (i, 0)),
            pl.BlockSpec((BOND_STEPS, NEIGH, H), lambda i: (0, 0, 0)),
            pl.BlockSpec((BOND_STEPS, NEIGH, H), lambda i: (0, 0, 0)),
            pl.BlockSpec((ANGLE_STEPS, NEIGH * NEIGH, H), lambda i: (0, 0, 0)),
            pl.BlockSpec((ANGLE_STEPS, NEIGH * NEIGH, H), lambda i: (0, 0, 0)),
        ],
        out_specs=[
            pl.BlockSpec((2, BLK, H), lambda i: (0, i, 0)),
            pl.BlockSpec((BLK, H), lambda i: (i, 0)),
            pl.BlockSpec((BLK, H), lambda i: (i, 0)),
        ],
        out_shape=[
            jax.ShapeDtypeStruct((2, NP, H), F32),
            jax.ShapeDtypeStruct((NP, H), F32),
            jax.ShapeDtypeStruct((NP, H), F32),
        ],
    )(bond_p, angle_p, wlb_s, wrb_s, wla_s, wra_s)


def _sc_mesh():
    return plsc.VectorSubcoreMesh(core_axis_name="c", subcore_axis_name="s",
                                  num_cores=NC, num_subcores=NS)


def _sc_scatter_body(z_hbm, idx_hbm, zeros_hbm, s_hbm, idx_v, z_v, sem, lsem,
                     hist):
    c = lax.axis_index("c")
    s = lax.axis_index("s")
    for p in range(NDSPLIT):
        span_h = pl.ds(s * SPANH, SPANH)
        pltpu.sync_copy(zeros_hbm.at[span_h], hist.at[span_h])
        pltpu.sync_copy(idx_hbm.at[p, pl.ds(s * NROW, NROW)], idx_v)
        plsc.subcore_barrier()
        ld = pltpu.async_copy(z_hbm.at[c, pl.ds(s * SPAN, CH)],
                              z_v.at[0], lsem)
        for j in range(NCH):
            ld.wait()
            descs = [
                pltpu.async_copy(
                    z_v.at[j % 2],
                    hist.at[plsc.Indices(idx_v.at[j * NEIGH + k],
                                         ignored_value=-1)],
                    sem, add=True)
                for k in range(NEIGH)
            ]
            if j + 1 < NCH:
                ld = pltpu.async_copy(
                    z_hbm.at[c, pl.ds(s * SPAN + (j + 1) * CH, CH)],
                    z_v.at[(j + 1) % 2], lsem)
            for dsc in descs:
                dsc.wait()
        plsc.subcore_barrier()
        pltpu.sync_copy(hist.at[span_h],
                        s_hbm.at[c, pl.ds(p * NPD + s * SPANH, SPANH)])


def _make_sc_scatter():
    return pl.kernel(
        _sc_scatter_body,
        out_type=jax.ShapeDtypeStruct((NC, NP, H), F32),
        mesh=_sc_mesh(),
        compiler_params=pltpu.CompilerParams(needs_layout_passes=False),
        scratch_types=[
            pltpu.VMEM((NROW, CH), jnp.int32),
            pltpu.VMEM((2, CH, H), F32),
            pltpu.SemaphoreType.DMA,
            pltpu.SemaphoreType.DMA,
            pltpu.VMEM_SHARED((NPD, H), F32),
        ],
    )


def _sc_cnt_body(idxr_hbm, zeros_hbm, ident_hbm, cnt_hbm,
                 idx_v, ident_v, hist2d, cnt_sh):
    s = lax.axis_index("s")
    zvec = jnp.zeros((16,), F32)
    ovec = jnp.ones((16,), F32)
    for i in range(NROW):
        for l in range(CH // 16):
            hist2d[i, pl.ds(l * 16, 16)] = zvec
    pltpu.sync_copy(idxr_hbm.at[pl.ds(s * NROW, NROW)], idx_v)
    pltpu.sync_copy(ident_hbm, ident_v)
    for i in range(NROW):
        for l in range(CH // 16):
            iv = idx_v[i, pl.ds(l * 16, 16)]
            hi = lax.shift_right_logical(iv, 7)
            lo = jnp.bitwise_and(iv, 127)
            plsc.addupdate_scatter(hist2d, [hi, lo], ovec)

    @pl.when(s == 0)
    def _():
        pltpu.sync_copy(zeros_hbm.at[pl.ds(0, NROW)], cnt_sh)
    plsc.subcore_barrier()
    for t in range(NS):
        @pl.when(s == t)
        def _():
            pltpu.sync_copy(hist2d, cnt_sh.at[ident_v], add=True)
        plsc.subcore_barrier()

    c = lax.axis_index("c")

    @pl.when(s == 0)
    def _():
        pltpu.sync_copy(cnt_sh, cnt_hbm.at[c])


def _make_sc_cnt():
    return pl.kernel(
        _sc_cnt_body,
        out_type=jax.ShapeDtypeStruct((NC, NROW, CH), F32),
        mesh=_sc_mesh(),
        compiler_params=pltpu.CompilerParams(needs_layout_passes=False),
        scratch_types=[
            pltpu.VMEM((NROW, CH), jnp.int32),
            pltpu.VMEM((NROW,), jnp.int32),
            pltpu.VMEM((NROW, CH), F32),
            pltpu.VMEM_SHARED((NROW, CH), F32),
        ],
    )


def _tc2_body(s_ref, cnt_ref, rb_ref, ra_ref,
              wlb_ref, wrb_ref, wla_ref, wra_ref,
              blb_ref, bla_ref, z_ref, r2b_ref, r2a_ref):
    inv = 1.0 / jnp.maximum(cnt_ref[...][:, 0:1], 1.0)
    b1 = jnp.maximum(s_ref[0] * inv + blb_ref[...] + rb_ref[...], 0.0)
    a1 = jnp.maximum(s_ref[1] * inv + bla_ref[...] + ra_ref[...], 0.0)
    z_ref[0] = jnp.dot(b1, wlb_ref[...], preferred_element_type=F32, precision=jax.lax.Precision.HIGHEST)
    r2b_ref[...] = jnp.dot(b1, wrb_ref[...], preferred_element_type=F32, precision=jax.lax.Precision.HIGHEST)
    z_ref[1] = jnp.dot(a1, wla_ref[...], preferred_element_type=F32, precision=jax.lax.Precision.HIGHEST)
    r2a_ref[...] = jnp.dot(a1, wra_ref[...], preferred_element_type=F32, precision=jax.lax.Precision.HIGHEST)


def _tc2(s1, cnt16, rb, ra, wlb_t, wrb_t, wla_t, wra_t, blb, bla):
    grid = (NP // BLK,)
    row_spec = pl.BlockSpec((BLK, H), lambda i: (i, 0))
    s_spec = pl.BlockSpec((2, BLK, H), lambda i: (0, i, 0))
    w_spec = pl.BlockSpec((H, H), lambda i: (0, 0))
    b_spec = pl.BlockSpec((1, H), lambda i: (0, 0))
    return pl.pallas_call(
        _tc2_body,
        grid=grid,
        in_specs=[s_spec,
                  pl.BlockSpec((BLK, NEIGH), lambda i: (i, 0)),
                  row_spec, row_spec, w_spec, w_spec, w_spec, w_spec,
                  b_spec, b_spec],
        out_specs=[s_spec, row_spec, row_spec],
        out_shape=[jax.ShapeDtypeStruct((2, NP, H), F32),
                   jax.ShapeDtypeStruct((NP, H), F32),
                   jax.ShapeDtypeStruct((NP, H), F32)],
    )(s1, cnt16, rb, ra, wlb_t, wrb_t, wla_t, wra_t, blb, bla)


def _tc3_body(s_ref, cnt_ref, rb_ref, ra_ref,
              blb_ref, bla_ref, wm1_ref, wm2_ref, bm_ref, wf_ref, bf_ref,
              out_ref):
    inv = 1.0 / jnp.maximum(cnt_ref[...][:, 0:1], 1.0)
    b2 = jnp.maximum(s_ref[0] * inv + blb_ref[...] + rb_ref[...], 0.0)
    a2 = jnp.maximum(s_ref[1] * inv + bla_ref[...] + ra_ref[...], 0.0)
    crys = (jnp.dot(b2, wm1_ref[...], preferred_element_type=F32, precision=jax.lax.Precision.HIGHEST)
            + jnp.dot(a2, wm2_ref[...], preferred_element_type=F32, precision=jax.lax.Precision.HIGHEST)
            + bm_ref[...])
    out_ref[...] = jnp.dot(crys, wf_ref[...],
                           preferred_element_type=F32, precision=jax.lax.Precision.HIGHEST) + bf_ref[...]


def _tc3(s2, cnt16, rb, ra, blb, bla, wm1, wm2, bm2d, wf_pad, bf_pad):
    grid = (NP // BLK,)
    row_spec = pl.BlockSpec((BLK, H), lambda i: (i, 0))
    return pl.pallas_call(
        _tc3_body,
        grid=grid,
        in_specs=[pl.BlockSpec((2, BLK, H), lambda i: (0, i, 0)),
                  pl.BlockSpec((BLK, NEIGH), lambda i: (i, 0)),
                  row_spec, row_spec,
                  pl.BlockSpec((1, H), lambda i: (0, 0)),
                  pl.BlockSpec((1, H), lambda i: (0, 0)),
                  pl.BlockSpec((H, 2 * H), lambda i: (0, 0)),
                  pl.BlockSpec((H, 2 * H), lambda i: (0, 0)),
                  pl.BlockSpec((1, 2 * H), lambda i: (0, 0)),
                  pl.BlockSpec((2 * H, H), lambda i: (0, 0)),
                  pl.BlockSpec((1, H), lambda i: (0, 0))],
        out_specs=[row_spec],
        out_shape=[jax.ShapeDtypeStruct((NP, H), F32)],
    )(s2, cnt16, rb, ra, blb, bla, wm1, wm2, bm2d, wf_pad, bf_pad)[0]


def kernel(bond_fea, angle_fea, species, nbr_idx, crys_idx,
           Wl1b, bl1b, Wr1b, Wl1a, bl1a, Wr1a,
           Wl2b, bl2b, Wr2b, Wl2a, bl2a, Wr2a,
           Wm, bm, Wf, bf):
    del species, crys_idx
    pad = NP - N
    bond_p = jnp.pad(bond_fea, ((0, pad), (0, 0)))
    angle_p = jnp.pad(angle_fea.reshape(N, NEIGH * NEIGH), ((0, pad), (0, 0)))

    wlb_s = jnp.transpose(Wl1b.T.reshape(NEIGH, BOND_STEPS, H), (1, 0, 2))
    wrb_s = jnp.transpose(Wr1b.T.reshape(NEIGH, BOND_STEPS, H), (1, 0, 2))
    wla_s = jnp.transpose(
        Wl1a.T.reshape(NEIGH * NEIGH, ANGLE_STEPS, H), (1, 0, 2))
    wra_s = jnp.transpose(
        Wr1a.T.reshape(NEIGH * NEIGH, ANGLE_STEPS, H), (1, 0, 2))

    nbr_ext = jnp.concatenate(
        [nbr_idx.astype(jnp.int32),
         jnp.full((pad, NEIGH), DUMP, jnp.int32)], axis=0)
    idx_raw = jnp.transpose(
        nbr_ext.T.reshape(NEIGH, NS, NCH, CH),
        (1, 2, 0, 3)).reshape(NS * NROW, CH)
    pad_mask = idx_raw >= N
    quarters = []
    for p in range(NDSPLIT):
        rel = idx_raw - p * NPD
        ok = (rel >= 0) & (rel < NPD) & ~pad_mask
        quarters.append(jnp.where(ok, rel, -1))
    idx_split = jnp.stack(quarters)

    zeros_hbm = jnp.zeros((NPD, H), F32)
    ident_hbm = jnp.arange(NROW, dtype=jnp.int32)

    cnt2 = _make_sc_cnt()(idx_raw, zeros_hbm, ident_hbm)
    cnt16 = jnp.broadcast_to(cnt2[0].reshape(NP, 1), (NP, NEIGH))

    z1, rb, ra = _tc1(bond_p, angle_p, wlb_s, wrb_s, wla_s, wra_s)

    s1 = _make_sc_scatter()(z1, idx_split, zeros_hbm)

    z2, r2b, r2a = _tc2(s1, cnt16, rb, ra,
                        Wl2b.T, Wr2b.T, Wl2a.T, Wr2a.T,
                        bl1b.reshape(1, H), bl1a.reshape(1, H))

    s2 = _make_sc_scatter()(z2, idx_split, zeros_hbm)

    wf_pad = jnp.pad(Wf.T, ((0, 0), (0, H - 2)))
    out_pad = _tc3(s2, cnt16, r2b, r2a,
                   bl2b.reshape(1, H), bl2a.reshape(1, H),
                   Wm.T[:H], Wm.T[H:], bm.reshape(1, 2 * H),
                   wf_pad, jnp.pad(bf, (0, H - 2)).reshape(1, H))
    return out_pad[:N, :2]
